# baseline (speedup 1.0000x reference)
import jax
import jax.numpy as jnp
from jax import lax
from jax.experimental import pallas as pl
from jax.experimental.pallas import tpu as pltpu
from jax.experimental.pallas import tpu_sc as plsc

B, L, H, Dn, Dh = 16, 2048, 24, 256, 100
LANES = 16
CHT = 64
NCH = L // (2 * CHT)
NTIL = NCH // 2
NBUF = 3

MASK_SC = -9999999999.0
MASK_WC = -99999999999.0
MASK_WV = -100000000000.0


def _full(v):
    return jnp.full((LANES,), v, jnp.int32)


def _body(wn, l_n_h, wh_h, l_hs_h, cls_h, g_sc_h, g_wc_h,
          o_sc, o_sa, o_wn, o_wc, o_wo, o_wv,
          gbuf, obuf, whb, clsb, lnb, lhsb, gscb, gwcb,
          scb, sab, wnb, wcb, wob, sem, sem2):
    b = lax.axis_index("s")
    h = lax.axis_index("c")
    iota = lax.iota(jnp.int32, LANES)

    def g0(ch):
        return (2 * (ch // 2) + h) * 128 + (ch % 2) * CHT

    def chunk_refs(ch):
        return (wn.at[b, pl.ds(g0(ch), CHT), pl.ds(0, Dn)],
                gbuf.at[ch % NBUF])

    def start(ch):
        s, d = chunk_refs(ch)
        pltpu.async_copy(s, d, sem)

    pltpu.sync_copy(l_n_h, lnb)
    pltpu.sync_copy(g_wc_h, gwcb)
    ln_b = plsc.load_gather(lnb, [_full(b)])
    myln = jnp.max(ln_b)

    for ch in range(NBUF):
        @pl.when(g0(ch) < myln)
        def _pro(ch=ch):
            start(ch)

    @pl.when(jnp.logical_and(b == 1, h == 0))
    def _small0():
        st_w = pltpu.async_copy(wh_h, whb, sem2)
        pltpu.sync_copy(l_hs_h, lhsb)
        st_w.wait()
        lhs_v = lhsb[...]
        for j in range(H):
            hm = jnp.int32(j) >= lhs_v
            v0 = plsc.load_gather(whb, [iota, _full(j), _full(0)])
            plsc.store_scatter(scb, [iota, _full(j)], jnp.where(hm, MASK_SC, v0))
            v8 = plsc.load_gather(whb, [iota, _full(j), _full(8)])
            plsc.store_scatter(wcb, [iota, _full(j)], jnp.where(hm, MASK_WC, v8))
        pltpu.sync_copy(scb, o_sc)
        pltpu.sync_copy(wcb, o_wc)

    @pl.when(jnp.logical_and(b == 1, h == 1))
    def _small1():
        st_w = pltpu.async_copy(wh_h, whb, sem2)
        pltpu.sync_copy(cls_h, clsb)
        pltpu.sync_copy(g_sc_h, gscb)
        st_w.wait()
        gsc_v = gscb[...]
        for j in range(6):
            v = plsc.load_gather(whb, [iota, gsc_v, _full(1 + j)])
            sab[j, pl.ds(0, LANES)] = v
        for j in range(5):
            v = plsc.load_gather(clsb, [iota, _full(j)])
            wnb[j, pl.ds(0, LANES)] = v
        for w in range(4):
            cw = plsc.load_gather(gwcb, [_full(w), iota])
            for j in range(4):
                v = plsc.load_gather(whb, [iota, cw, _full(10 + j)])
                wob[w, j, pl.ds(0, LANES)] = v
        pltpu.sync_copy(sab, o_sa)
        pltpu.sync_copy(wnb, o_wn)
        pltpu.sync_copy(wob, o_wo)

    chans = []
    for w in range(4):
        c0 = plsc.load_gather(gwcb, [_full(w), _full(b)])
        chans.append((c0, c0 + 100))

    mvec = jnp.full((LANES,), jnp.float32(MASK_WV), jnp.float32)
    for ch in range(NCH):
        tj, half = ch // 2, ch % 2

        @pl.when(g0(ch) < myln)
        def _work(ch=ch, tj=tj, half=half):
            s, d = chunk_refs(ch)
            pltpu.make_async_copy(s, d, sem).wait()
            cur = _full(ch % NBUF)

            def mloop(m, _, cur=cur, ch=ch, tj=tj, half=half):
                l_loc = m * LANES + iota
                nm = (g0(ch) + l_loc) >= ln_b
                for w in range(4):
                    for k in range(2):
                        vals = plsc.load_gather(gbuf, [cur, l_loc, chans[w][k]])
                        vals = jnp.where(nm, MASK_WV, vals)
                        obuf[w, tj, k,
                             pl.ds(half * CHT + m * LANES, LANES)] = vals
                return 0

            lax.fori_loop(0, CHT // LANES, mloop, 0)

        @pl.when(g0(ch) >= myln)
        def _fill(ch=ch, tj=tj, half=half):
            def floop(m, _, tj=tj, half=half):
                for w in range(4):
                    for k in range(2):
                        obuf[w, tj, k,
                             pl.ds(half * CHT + m * LANES, LANES)] = mvec
                return 0

            lax.fori_loop(0, CHT // LANES, floop, 0)

        if ch + NBUF < NCH:
            @pl.when(g0(ch + NBUF) < myln)
            def _nxt(ch=ch):
                start(ch + NBUF)

    writes = [
        pltpu.async_copy(obuf.at[w],
                         o_wv.at[b * 4 + w, pl.ds(0, NTIL), h], sem2)
        for w in range(4)
    ]
    for c in writes:
        c.wait()


def _sc_call(wn, l_n, wh, l_hs, cls, g_sc, g_wc_t):
    return pl.kernel(
        _body,
        out_type=[
            jax.ShapeDtypeStruct((B, H), jnp.float32),
            jax.ShapeDtypeStruct((6, B), jnp.float32),
            jax.ShapeDtypeStruct((5, B), jnp.float32),
            jax.ShapeDtypeStruct((B, H), jnp.float32),
            jax.ShapeDtypeStruct((4, 4, B), jnp.float32),
            jax.ShapeDtypeStruct((B * 4, NTIL, 2, 2, 128), jnp.float32),
        ],
        mesh=plsc.VectorSubcoreMesh(core_axis_name="c", subcore_axis_name="s"),
        compiler_params=pltpu.CompilerParams(needs_layout_passes=False),
        scratch_types=[
            pltpu.VMEM((NBUF, CHT, Dn), jnp.float32),
            pltpu.VMEM((4, NTIL, 2, 128), jnp.float32),
            pltpu.VMEM((B, H, Dh), jnp.float32),
            pltpu.VMEM((B, Dh), jnp.float32),
            pltpu.VMEM((B,), jnp.int32),
            pltpu.VMEM((B,), jnp.int32),
            pltpu.VMEM((B,), jnp.int32),
            pltpu.VMEM((4, B), jnp.int32),
            pltpu.VMEM((B, H), jnp.float32),
            pltpu.VMEM((6, B), jnp.float32),
            pltpu.VMEM((5, B), jnp.float32),
            pltpu.VMEM((B, H), jnp.float32),
            pltpu.VMEM((4, 4, B), jnp.float32),
            pltpu.SemaphoreType.DMA,
            pltpu.SemaphoreType.DMA,
        ],
    )(wn, l_n, wh, l_hs, cls, g_sc, g_wc_t)


def kernel(wemb_n, l_n, wemb_h, l_hs, cls_vec, g_sc, g_sa, g_wn, g_wc, g_wo):
    o_sc, o_sa, o_wn, o_wc, o_wo, o_wv = _sc_call(
        wemb_n,
        l_n.astype(jnp.int32),
        wemb_h,
        l_hs.astype(jnp.int32),
        cls_vec,
        g_sc.astype(jnp.int32),
        g_wc.astype(jnp.int32).T,
    )
    s_wv = (o_wv.reshape(B, 4, NTIL, 2, 2, 128)
            .transpose(0, 1, 2, 3, 5, 4)
            .reshape(B, 4, L, 2))
    return (o_sc, o_sa.T, o_wn.T, o_wc, jnp.transpose(o_wo, (2, 0, 1)), s_wv)


# speedup vs baseline: 1.0973x; 1.0973x over previous
import jax
import jax.numpy as jnp
from jax import lax
from jax.experimental import pallas as pl
from jax.experimental.pallas import tpu as pltpu
from jax.experimental.pallas import tpu_sc as plsc

B, L, H, Dn, Dh = 16, 2048, 24, 256, 100
LANES = 16
CHT = 64
NCH = L // (2 * CHT)
NTIL = NCH // 2
NBUF = 4
CW = 128

MASK_SC = -9999999999.0
MASK_WC = -99999999999.0
MASK_WV = -100000000000.0


def _full(v):
    return jnp.full((LANES,), v, jnp.int32)


def _body(wn, l_n_h, wh_h, l_hs_h, cls_h, g_sc_h, g_wc_h,
          o_sc, o_sa, o_wn, o_wc, o_wo, o_wv,
          gbuf, obuf, whb, clsb, lnb, lhsb, gscb, gwcb,
          scb, sab, wnb, wcb, wob, sem, sem2):
    b = lax.axis_index("s")
    h = lax.axis_index("c")
    iota = lax.iota(jnp.int32, LANES)

    def g0(ch):
        return (2 * (ch // 2) + h) * 128 + (ch % 2) * CHT

    def chunk_refs(ch):
        return (wn.at[b, pl.ds(g0(ch), CHT), pl.ds(0, CW)],
                gbuf.at[ch % NBUF])

    def start(ch):
        s, d = chunk_refs(ch)
        pltpu.async_copy(s, d, sem)

    pltpu.sync_copy(l_n_h, lnb)
    pltpu.sync_copy(g_wc_h, gwcb)
    ln_b = plsc.load_gather(lnb, [_full(b)])
    myln = jnp.max(ln_b)

    for ch in range(NBUF):
        @pl.when(g0(ch) < myln)
        def _pro(ch=ch):
            start(ch)

    @pl.when(jnp.logical_and(b == 1, h == 0))
    def _small0():
        st_w = pltpu.async_copy(wh_h, whb, sem2)
        pltpu.sync_copy(l_hs_h, lhsb)
        st_w.wait()
        lhs_v = lhsb[...]
        for j in range(H):
            hm = jnp.int32(j) >= lhs_v
            v0 = plsc.load_gather(whb, [iota, _full(j), _full(0)])
            plsc.store_scatter(scb, [iota, _full(j)], jnp.where(hm, MASK_SC, v0))
            v8 = plsc.load_gather(whb, [iota, _full(j), _full(8)])
            plsc.store_scatter(wcb, [iota, _full(j)], jnp.where(hm, MASK_WC, v8))
        pltpu.sync_copy(scb, o_sc)
        pltpu.sync_copy(wcb, o_wc)

    @pl.when(jnp.logical_and(b == 1, h == 1))
    def _small1():
        st_w = pltpu.async_copy(wh_h, whb, sem2)
        pltpu.sync_copy(cls_h, clsb)
        pltpu.sync_copy(g_sc_h, gscb)
        st_w.wait()
        gsc_v = gscb[...]
        for j in range(6):
            v = plsc.load_gather(whb, [iota, gsc_v, _full(1 + j)])
            sab[j, pl.ds(0, LANES)] = v
        for j in range(5):
            v = plsc.load_gather(clsb, [iota, _full(j)])
            wnb[j, pl.ds(0, LANES)] = v
        for w in range(4):
            cw = plsc.load_gather(gwcb, [_full(w), iota])
            for j in range(4):
                v = plsc.load_gather(whb, [iota, cw, _full(10 + j)])
                wob[w, j, pl.ds(0, LANES)] = v
        pltpu.sync_copy(sab, o_sa)
        pltpu.sync_copy(wnb, o_wn)
        pltpu.sync_copy(wob, o_wo)

    chans = []
    for w in range(4):
        c0 = plsc.load_gather(gwcb, [_full(w), _full(b)])
        chans.append((c0, c0 + 100))

    mvec = jnp.full((LANES,), jnp.float32(MASK_WV), jnp.float32)
    for ch in range(NCH):
        tj, half = ch // 2, ch % 2

        @pl.when(g0(ch) < myln)
        def _work(ch=ch, tj=tj, half=half):
            s, d = chunk_refs(ch)
            pltpu.make_async_copy(s, d, sem).wait()
            cur = _full(ch % NBUF)

            def mloop(m, _, cur=cur, ch=ch, tj=tj, half=half):
                l_loc = m * LANES + iota
                nm = (g0(ch) + l_loc) >= ln_b
                for w in range(4):
                    for k in range(2):
                        vals = plsc.load_gather(gbuf, [cur, l_loc, chans[w][k]])
                        vals = jnp.where(nm, MASK_WV, vals)
                        obuf[w, tj, k,
                             pl.ds(half * CHT + m * LANES, LANES)] = vals
                return 0

            lax.fori_loop(0, CHT // LANES, mloop, 0)

        @pl.when(g0(ch) >= myln)
        def _fill(ch=ch, tj=tj, half=half):
            def floop(m, _, tj=tj, half=half):
                for w in range(4):
                    for k in range(2):
                        obuf[w, tj, k,
                             pl.ds(half * CHT + m * LANES, LANES)] = mvec
                return 0

            lax.fori_loop(0, CHT // LANES, floop, 0)

        if ch + NBUF < NCH:
            @pl.when(g0(ch + NBUF) < myln)
            def _nxt(ch=ch):
                start(ch + NBUF)

    writes = [
        pltpu.async_copy(obuf.at[w],
                         o_wv.at[b * 4 + w, pl.ds(0, NTIL), h], sem2)
        for w in range(4)
    ]
    for c in writes:
        c.wait()


def _sc_call(wn, l_n, wh, l_hs, cls, g_sc, g_wc_t):
    return pl.kernel(
        _body,
        out_type=[
            jax.ShapeDtypeStruct((B, H), jnp.float32),
            jax.ShapeDtypeStruct((6, B), jnp.float32),
            jax.ShapeDtypeStruct((5, B), jnp.float32),
            jax.ShapeDtypeStruct((B, H), jnp.float32),
            jax.ShapeDtypeStruct((4, 4, B), jnp.float32),
            jax.ShapeDtypeStruct((B * 4, NTIL, 2, 2, 128), jnp.float32),
        ],
        mesh=plsc.VectorSubcoreMesh(core_axis_name="c", subcore_axis_name="s"),
        compiler_params=pltpu.CompilerParams(needs_layout_passes=False),
        scratch_types=[
            pltpu.VMEM((NBUF, CHT, CW), jnp.float32),
            pltpu.VMEM((4, NTIL, 2, 128), jnp.float32),
            pltpu.VMEM((B, H, Dh), jnp.float32),
            pltpu.VMEM((B, Dh), jnp.float32),
            pltpu.VMEM((B,), jnp.int32),
            pltpu.VMEM((B,), jnp.int32),
            pltpu.VMEM((B,), jnp.int32),
            pltpu.VMEM((4, B), jnp.int32),
            pltpu.VMEM((B, H), jnp.float32),
            pltpu.VMEM((6, B), jnp.float32),
            pltpu.VMEM((5, B), jnp.float32),
            pltpu.VMEM((B, H), jnp.float32),
            pltpu.VMEM((4, 4, B), jnp.float32),
            pltpu.SemaphoreType.DMA,
            pltpu.SemaphoreType.DMA,
        ],
    )(wn, l_n, wh, l_hs, cls, g_sc, g_wc_t)


def kernel(wemb_n, l_n, wemb_h, l_hs, cls_vec, g_sc, g_sa, g_wn, g_wc, g_wo):
    o_sc, o_sa, o_wn, o_wc, o_wo, o_wv = _sc_call(
        wemb_n,
        l_n.astype(jnp.int32),
        wemb_h,
        l_hs.astype(jnp.int32),
        cls_vec,
        g_sc.astype(jnp.int32),
        g_wc.astype(jnp.int32).T,
    )
    s_wv = (o_wv.reshape(B, 4, NTIL, 2, 2, 128)
            .transpose(0, 1, 2, 3, 5, 4)
            .reshape(B, 4, L, 2))
    return (o_sc, o_sa.T, o_wn.T, o_wc, jnp.transpose(o_wo, (2, 0, 1)), s_wv)


# speedup vs baseline: 1.1679x; 1.0643x over previous
import jax
import jax.numpy as jnp
from jax import lax
from jax.experimental import pallas as pl
from jax.experimental.pallas import tpu as pltpu
from jax.experimental.pallas import tpu_sc as plsc

B, L, H, Dn, Dh = 16, 2048, 24, 256, 100
LANES = 16
HALF = L // 2
CHT = 128
NCH = HALF // CHT
CW = 128
NBUF = 3

MASK_SC = -9999999999.0
MASK_WC = -99999999999.0
MASK_WV = -100000000000.0


def _full(v):
    return jnp.full((LANES,), v, jnp.int32)


def _body(wn, l_n_h, wh_h, l_hs_h, cls_h, g_sc_h, g_wc_h,
          o_sc, o_sa, o_wn, o_wc, o_wo, o_wv,
          gbuf, obuf, whb, clsb, lnb, lhsb, gscb, gwcb,
          scb, sab, wnb, wcb, wob, sem, sem2):
    b = lax.axis_index("s")
    h = lax.axis_index("c")
    iota = lax.iota(jnp.int32, LANES)

    def chunk_refs(ch):
        return (wn.at[b, pl.ds((2 * ch + h) * CHT, CHT), pl.ds(0, CW)],
                gbuf.at[ch % NBUF])

    def start(ch):
        s, d = chunk_refs(ch)
        pltpu.async_copy(s, d, sem)

    pltpu.sync_copy(l_n_h, lnb)
    pltpu.sync_copy(g_wc_h, gwcb)
    ln_b = plsc.load_gather(lnb, [_full(b)])

    myln = jnp.max(ln_b)
    nt = lax.div(myln + (CHT - 1), CHT)
    nchw = jnp.clip(lax.div(nt - h + 1, 2), 0, NCH)

    for ch in range(NBUF):
        @pl.when(ch < nchw)
        def _pro(ch=ch):
            start(ch)

    @pl.when(jnp.logical_and(b == 1, h == 0))
    def _small0():
        st_w = pltpu.async_copy(wh_h, whb, sem2)
        pltpu.sync_copy(l_hs_h, lhsb)
        st_w.wait()
        lhs_v = lhsb[...]
        for j in range(H):
            hm = jnp.int32(j) >= lhs_v
            v0 = plsc.load_gather(whb, [iota, _full(j), _full(0)])
            plsc.store_scatter(scb, [iota, _full(j)], jnp.where(hm, MASK_SC, v0))
            v8 = plsc.load_gather(whb, [iota, _full(j), _full(8)])
            plsc.store_scatter(wcb, [iota, _full(j)], jnp.where(hm, MASK_WC, v8))
        pltpu.sync_copy(scb, o_sc)
        pltpu.sync_copy(wcb, o_wc)

    @pl.when(jnp.logical_and(b == 1, h == 1))
    def _small1():
        st_w = pltpu.async_copy(wh_h, whb, sem2)
        pltpu.sync_copy(cls_h, clsb)
        pltpu.sync_copy(g_sc_h, gscb)
        st_w.wait()
        gsc_v = gscb[...]
        for j in range(6):
            v = plsc.load_gather(whb, [iota, gsc_v, _full(1 + j)])
            sab[j, pl.ds(0, LANES)] = v
        for j in range(5):
            v = plsc.load_gather(clsb, [iota, _full(j)])
            wnb[j, pl.ds(0, LANES)] = v
        for w in range(4):
            cw = plsc.load_gather(gwcb, [_full(w), iota])
            for j in range(4):
                v = plsc.load_gather(whb, [iota, cw, _full(10 + j)])
                wob[w, j, pl.ds(0, LANES)] = v
        pltpu.sync_copy(sab, o_sa)
        pltpu.sync_copy(wnb, o_wn)
        pltpu.sync_copy(wob, o_wo)

    chans = []
    for w in range(4):
        c0 = plsc.load_gather(gwcb, [_full(w), _full(b)])
        chans.append((c0, c0 + 100))

    mvec = jnp.full((LANES,), jnp.float32(MASK_WV), jnp.float32)
    for ch in range(NCH):
        @pl.when(ch < nchw)
        def _work(ch=ch):
            s, d = chunk_refs(ch)
            pltpu.make_async_copy(s, d, sem).wait()
            cur = _full(ch % NBUF)

            def mloop(m, _, cur=cur, ch=ch):
                l_loc = m * LANES + iota
                nm = ((2 * ch + h) * CHT + l_loc) >= ln_b
                for w in range(4):
                    for k in range(2):
                        vals = plsc.load_gather(gbuf, [cur, l_loc, chans[w][k]])
                        vals = jnp.where(nm, MASK_WV, vals)
                        obuf[w, ch, k, pl.ds(m * LANES, LANES)] = vals
                return 0

            lax.fori_loop(0, CHT // LANES, mloop, 0)

        @pl.when(ch >= nchw)
        def _fill(ch=ch):
            def floop(m, _, ch=ch):
                for w in range(4):
                    for k in range(2):
                        obuf[w, ch, k, pl.ds(m * LANES, LANES)] = mvec
                return 0

            lax.fori_loop(0, CHT // LANES, floop, 0)

        if ch + NBUF < NCH:
            @pl.when(ch + NBUF < nchw)
            def _nxt(ch=ch):
                start(ch + NBUF)

    writes = [
        pltpu.async_copy(obuf.at[w],
                         o_wv.at[b * 4 + w, pl.ds(0, NCH), h], sem2)
        for w in range(4)
    ]
    for c in writes:
        c.wait()


def _sc_call(wn, l_n, wh, l_hs, cls, g_sc, g_wc_t):
    return pl.kernel(
        _body,
        out_type=[
            jax.ShapeDtypeStruct((B, H), jnp.float32),
            jax.ShapeDtypeStruct((6, B), jnp.float32),
            jax.ShapeDtypeStruct((5, B), jnp.float32),
            jax.ShapeDtypeStruct((B, H), jnp.float32),
            jax.ShapeDtypeStruct((4, 4, B), jnp.float32),
            jax.ShapeDtypeStruct((B * 4, NCH, 2, 2, CHT), jnp.float32),
        ],
        mesh=plsc.VectorSubcoreMesh(core_axis_name="c", subcore_axis_name="s"),
        compiler_params=pltpu.CompilerParams(needs_layout_passes=False),
        scratch_types=[
            pltpu.VMEM((NBUF, CHT, CW), jnp.float32),
            pltpu.VMEM((4, NCH, 2, CHT), jnp.float32),
            pltpu.VMEM((B, H, Dh), jnp.float32),
            pltpu.VMEM((B, Dh), jnp.float32),
            pltpu.VMEM((B,), jnp.int32),
            pltpu.VMEM((B,), jnp.int32),
            pltpu.VMEM((B,), jnp.int32),
            pltpu.VMEM((4, B), jnp.int32),
            pltpu.VMEM((B, H), jnp.float32),
            pltpu.VMEM((6, B), jnp.float32),
            pltpu.VMEM((5, B), jnp.float32),
            pltpu.VMEM((B, H), jnp.float32),
            pltpu.VMEM((4, 4, B), jnp.float32),
            pltpu.SemaphoreType.DMA,
            pltpu.SemaphoreType.DMA,
        ],
    )(wn, l_n, wh, l_hs, cls, g_sc, g_wc_t)


def kernel(wemb_n, l_n, wemb_h, l_hs, cls_vec, g_sc, g_sa, g_wn, g_wc, g_wo):
    o_sc, o_sa, o_wn, o_wc, o_wo, o_wv = _sc_call(
        wemb_n,
        l_n.astype(jnp.int32),
        wemb_h,
        l_hs.astype(jnp.int32),
        cls_vec,
        g_sc.astype(jnp.int32),
        g_wc.astype(jnp.int32).T,
    )
    s_wv = (o_wv.reshape(B, 4, NCH, 2, 2, CHT)
            .transpose(0, 1, 2, 3, 5, 4)
            .reshape(B, 4, L, 2))
    return (o_sc, o_sa.T, o_wn.T, o_wc, jnp.transpose(o_wo, (2, 0, 1)), s_wv)
